# baseline (speedup 1.0000x reference)
import jax
import jax.numpy as jnp
import numpy as np
from jax.experimental import pallas as pl
from jax.experimental.pallas import tpu as pltpu

_TAU = 0.01
_N_EDGES = 4
_BR = 8
_CW = 2048
_TINY = np.float32(1.1754943508222875e-38)
_BIG = np.int32(2**30)

_ROT = ((13, 15, 26, 6), (17, 29, 16, 24))


def _gumbel_chunk(cnt, k0, k1, kx):
    v0 = jnp.zeros(cnt.shape, jnp.uint32) + k0
    v1 = cnt + k1
    ks = (k0, k1, kx)
    for grp in range(5):
        for r in _ROT[grp % 2]:
            v0 = v0 + v1
            v1 = (v1 << r) | (v1 >> (32 - r))
            v1 = v0 ^ v1
        v0 = v0 + ks[(grp + 1) % 3]
        v1 = v1 + (ks[(grp + 2) % 3] + jnp.uint32(grp + 1))
    bits = v0 ^ v1
    mant = (bits >> 9) | jnp.uint32(0x3F800000)
    floats = jax.lax.bitcast_convert_type(mant, jnp.float32) - jnp.float32(1.0)
    u = jnp.maximum(_TINY, floats + _TINY)
    return -jnp.log(-jnp.log(u))


def _sample_body(tgt_ref, keys_ref, w_ref, out_ref):
    num_targets = w_ref.shape[1]
    tgt = tgt_ref[0]
    rows = jax.lax.broadcasted_iota(jnp.int32, (_BR, _CW), 0)
    cols = jax.lax.broadcasted_iota(jnp.int32, (_BR, _CW), 1)
    row0 = ((pl.program_id(0) * _BR + rows) * num_targets).astype(jnp.uint32)
    row_base = row0 + cols.astype(jnp.uint32)
    row0_i = row0[:, :1].astype(jnp.int32)
    tgt_cnt = row0_i + tgt

    n_main = num_targets // _CW
    tail_base = n_main * _CW
    tail = num_targets - tail_base

    samp_cnts = []
    for it in range(_N_EDGES - 1):
        k0 = keys_ref[2 * it].astype(jnp.uint32)
        k1 = keys_ref[2 * it + 1].astype(jnp.uint32)
        kx = k0 ^ k1 ^ jnp.uint32(0x1BD11BDA)

        def scan_chunk(w_chunk, base, acc_max, acc_cnt):
            width = w_chunk.shape[1]
            cnt = row_base[:, :width] + jnp.uint32(0) + base
            g = _gumbel_chunk(cnt, k0, k1, kx)
            cnt_i = cnt.astype(jnp.int32)
            masked = cnt_i == tgt_cnt
            for s in samp_cnts:
                masked = masked | (cnt_i == s)
            s_val = w_chunk + g
            upd = (s_val > acc_max) & jnp.logical_not(masked)
            return (jnp.where(upd, s_val, acc_max),
                    jnp.where(upd, cnt_i, acc_cnt))

        def body(c, carry):
            base = (c * _CW).astype(jnp.uint32)
            return scan_chunk(w_ref[:, pl.ds(c * _CW, _CW)], base, *carry)

        init = (jnp.full((_BR, _CW), -jnp.inf, jnp.float32),
                jnp.zeros((_BR, _CW), jnp.int32))
        acc_max, acc_cnt = jax.lax.fori_loop(0, n_main, body, init)

        m = jnp.max(acc_max, axis=1, keepdims=True)
        win_parts = [jnp.where(acc_max >= m, acc_cnt, _BIG)]
        if tail:
            t_init = (jnp.full((_BR, tail), -jnp.inf, jnp.float32),
                      jnp.zeros((_BR, tail), jnp.int32))
            t_max, t_cnt = scan_chunk(
                w_ref[:, tail_base:num_targets], jnp.uint32(tail_base),
                *t_init)
            tm = jnp.max(t_max, axis=1, keepdims=True)
            m2 = jnp.maximum(m, tm)
            win_parts = [
                jnp.where((acc_max >= m2), acc_cnt, _BIG),
                jnp.where((t_max >= m2), t_cnt, _BIG),
            ]
        win = jnp.minimum(
            jnp.min(win_parts[0], axis=1, keepdims=True),
            jnp.min(win_parts[1], axis=1, keepdims=True)
            if len(win_parts) > 1 else _BIG)
        samp_cnts.append(win)

    out_ref[:, 0] = jnp.full((_BR,), tgt, jnp.float32)
    for it, s in enumerate(samp_cnts):
        out_ref[:, it + 1] = (s - row0_i)[:, 0].astype(jnp.float32)


def kernel(sample_weight, target_idx):
    num_nodes, num_targets = sample_weight.shape
    skey = jax.random.key(42)
    keys = jnp.concatenate([
        jax.random.key_data(jax.random.fold_in(skey, i))
        for i in range(_N_EDGES - 1)
    ]).astype(jnp.int32)
    tgt = jnp.asarray(target_idx, jnp.int32).reshape(1)

    out = pl.pallas_call(
        _sample_body,
        grid_spec=pltpu.PrefetchScalarGridSpec(
            num_scalar_prefetch=2,
            grid=(num_nodes // _BR,),
            in_specs=[pl.BlockSpec((_BR, num_targets), lambda i, *_: (i, 0))],
            out_specs=pl.BlockSpec((_BR, _N_EDGES), lambda i, *_: (i, 0)),
        ),
        out_shape=jax.ShapeDtypeStruct((num_nodes, _N_EDGES), jnp.float32),
    )(tgt, keys, sample_weight)
    return out
